# baseline (speedup 1.0000x reference)
import functools

import jax
import jax.numpy as jnp
from jax import lax
from jax.experimental import pallas as pl
from jax.experimental.pallas import tpu as pltpu
from jax.experimental.pallas import tpu_sc as plsc

NODE_DIM = 128
N_NODES = 10000
N_EDGES = 320000

NSUB = 16
EPT = N_EDGES // NSUB
VPT = EPT // 16
NP = 10240
NPAD = 10240
CHUNK = NP // NSUB


_PROJ_BLK = 1024


def _proj_body(w_ref, b_ref, x_ref, o_ref):
    i = pl.program_id(0)
    wc = jnp.concatenate(
        [w_ref[0:1, :NODE_DIM], w_ref[0:1, NODE_DIM:],
         w_ref[1:2, :NODE_DIM], w_ref[1:2, NODE_DIM:]], axis=0)
    proj = lax.dot_general(
        wc, x_ref[...], (((1,), (1,)), ((), ())),
        preferred_element_type=jnp.float32,
    )
    zero1 = jnp.zeros((1, 1), jnp.float32)
    bias = jnp.concatenate(
        [b_ref[0:1, 0:1], zero1, b_ref[0:1, 1:2], zero1], axis=0)
    o_ref[:, pl.ds(i * _PROJ_BLK, _PROJ_BLK)] = proj + bias


def _projection(x, W, b):
    return pl.pallas_call(
        _proj_body,
        grid=(NPAD // _PROJ_BLK,),
        in_specs=[
            pl.BlockSpec((2, 2 * NODE_DIM), lambda i: (0, 0)),
            pl.BlockSpec((1, 2), lambda i: (0, 0)),
            pl.BlockSpec((_PROJ_BLK, NODE_DIM), lambda i: (i, 0)),
        ],
        out_specs=pl.BlockSpec((4, NPAD), lambda i: (0, 0)),
        out_shape=jax.ShapeDtypeStruct((4, NPAD), jnp.float32),
    )(W, b.reshape(1, 2), x)


def _sc_body(tab_hbm, eidx_hbm,
             v0_hbm, v1_hbm,
             tab_v, row_v, col_v, ex_v, den_v,
             part_sh, inv_sh,
             sem0, sem1, sem2):
    cid = lax.axis_index("c")
    sid = lax.axis_index("s")
    base = sid * EPT
    off_a = cid * (2 * NPAD)

    d0 = pltpu.async_copy(tab_hbm.at[pl.ds(off_a, 2 * NPAD)], tab_v, sem0)
    d1 = pltpu.async_copy(eidx_hbm.at[pl.ds(base, EPT)], row_v, sem1)
    d2 = pltpu.async_copy(eidx_hbm.at[pl.ds(N_EDGES + base, EPT)], col_v, sem2)

    zeros = jnp.zeros((16,), jnp.float32)

    @plsc.parallel_loop(0, NP // 16, unroll=8)
    def _zero(i):
        den_v[pl.ds(i * 16, 16)] = zeros

    d0.wait()
    d1.wait()
    d2.wait()

    @plsc.parallel_loop(0, VPT, unroll=8)
    def _stepa(i):
        sl = pl.ds(i * 16, 16)
        r = row_v[sl]
        cl = col_v[sl]
        a = plsc.load_gather(tab_v, [r])
        c = plsc.load_gather(tab_v, [cl + NPAD])
        s = jnp.minimum(jnp.maximum(a + c, 0.0), 80.0)
        e = jnp.exp(s)
        ex_v[sl] = e
        plsc.addupdate_scatter(den_v, [r], e)

    pltpu.sync_copy(den_v, part_sh.at[pl.ds(sid * NP, NP)])
    plsc.subcore_barrier()

    cb = sid * CHUNK

    def _gather_parts(j, _):
        pltpu.sync_copy(part_sh.at[pl.ds(j * NP + cb, CHUNK)],
                        tab_v.at[pl.ds(j * CHUNK, CHUNK)])
        return ()

    lax.fori_loop(0, NSUB, _gather_parts, ())

    eps = jnp.float32(1e-16)

    @plsc.parallel_loop(0, CHUNK // 16, unroll=4)
    def _reduce(i):
        acc = tab_v[pl.ds(i * 16, 16)]

        def _acc(j, a_):
            return a_ + tab_v[pl.ds(j * CHUNK + i * 16, 16)]

        acc = lax.fori_loop(1, NSUB, _acc, acc)
        den_v[pl.ds(i * 16, 16)] = 1.0 / (acc + eps)

    pltpu.sync_copy(den_v.at[pl.ds(0, CHUNK)], inv_sh.at[pl.ds(cb, CHUNK)])
    plsc.subcore_barrier()
    pltpu.sync_copy(inv_sh, den_v)

    @plsc.parallel_loop(0, VPT, unroll=8)
    def _stepb(i):
        sl = pl.ds(i * 16, 16)
        inv = plsc.load_gather(den_v, [row_v[sl]])
        ex_v[sl] = ex_v[sl] * inv

    @pl.when(cid == 0)
    def _():
        pltpu.sync_copy(ex_v, v0_hbm.at[pl.ds(base, EPT)])

    @pl.when(cid == 1)
    def _():
        pltpu.sync_copy(ex_v, v1_hbm.at[pl.ds(base, EPT)])


@functools.partial(
    pl.kernel,
    out_type=(
        jax.ShapeDtypeStruct((N_EDGES,), jnp.float32),
        jax.ShapeDtypeStruct((N_EDGES,), jnp.float32),
    ),
    mesh=plsc.VectorSubcoreMesh(core_axis_name="c", subcore_axis_name="s"),
    scratch_types=[
        pltpu.VMEM((2 * NPAD,), jnp.float32),
        pltpu.VMEM((EPT,), jnp.int32),
        pltpu.VMEM((EPT,), jnp.int32),
        pltpu.VMEM((EPT,), jnp.float32),
        pltpu.VMEM((NP,), jnp.float32),
        pltpu.VMEM_SHARED((NSUB * NP,), jnp.float32),
        pltpu.VMEM_SHARED((NP,), jnp.float32),
        pltpu.SemaphoreType.DMA,
        pltpu.SemaphoreType.DMA,
        pltpu.SemaphoreType.DMA,
    ],
    compiler_params=pltpu.CompilerParams(needs_layout_passes=False),
)
def _sc_fused(*refs):
    _sc_body(*refs)


def _combine_body(v0_ref, v1_ref, amt_ref, o_ref):
    o_ref[...] = jnp.where(amt_ref[...] != 0, v0_ref[...] + v1_ref[...], 0.0)


def _combine(v0, v1, amt):
    blk = 32768
    return pl.pallas_call(
        _combine_body,
        grid=(pl.cdiv(N_EDGES, blk),),
        in_specs=[
            pl.BlockSpec((blk,), lambda i: (i,)),
            pl.BlockSpec((blk,), lambda i: (i,)),
            pl.BlockSpec((blk,), lambda i: (i,)),
        ],
        out_specs=pl.BlockSpec((blk,), lambda i: (i,)),
        out_shape=jax.ShapeDtypeStruct((N_EDGES,), jnp.float32),
    )(v0, v1, amt)


def kernel(x, W, b, edge_index, actual_amount):
    eidx = edge_index.astype(jnp.int32).reshape(2 * N_EDGES)
    amt = actual_amount.astype(jnp.int32)

    tab = _projection(x, W, b).reshape(-1)

    v0, v1 = _sc_fused(tab, eidx)
    return _combine(v0, v1, amt)


# speedup vs baseline: 1.1928x; 1.1928x over previous
import functools

import jax
import jax.numpy as jnp
from jax import lax
from jax.experimental import pallas as pl
from jax.experimental.pallas import tpu as pltpu
from jax.experimental.pallas import tpu_sc as plsc

NODE_DIM = 128
N_NODES = 10000
N_EDGES = 320000

NSUB = 16
EPT = N_EDGES // NSUB
VPT = EPT // 16
NP = 10240
CHUNK = NP // NSUB


def _proj_body(w_ref, b_ref, x_ref, o_ref):
    wc = jnp.concatenate(
        [w_ref[0:1, :NODE_DIM], w_ref[0:1, NODE_DIM:],
         w_ref[1:2, :NODE_DIM], w_ref[1:2, NODE_DIM:]], axis=0)
    proj = lax.dot_general(
        wc, x_ref[...], (((1,), (1,)), ((), ())),
        preferred_element_type=jnp.float32,
    )
    zero1 = jnp.zeros((1, 1), jnp.float32)
    bias = jnp.concatenate(
        [b_ref[0:1, 0:1], zero1, b_ref[0:1, 1:2], zero1], axis=0)
    o_ref[...] = proj + bias


def _projection(x, W, b):
    return pl.pallas_call(
        _proj_body,
        out_shape=jax.ShapeDtypeStruct((4, N_NODES), jnp.float32),
    )(W, b.reshape(1, 2), x)


def _sc_body(tab_hbm, eidx_hbm,
             v0_hbm, v1_hbm,
             tab_v, row_v, col_v, ex_v, den_v,
             part_sh, inv_sh,
             sem0, sem1, sem2):
    cid = lax.axis_index("c")
    sid = lax.axis_index("s")
    base = sid * EPT
    off_a = cid * (2 * N_NODES)

    d0 = pltpu.async_copy(tab_hbm.at[pl.ds(off_a, 2 * N_NODES)], tab_v, sem0)
    d1 = pltpu.async_copy(eidx_hbm.at[pl.ds(base, EPT)], row_v, sem1)
    d2 = pltpu.async_copy(eidx_hbm.at[pl.ds(N_EDGES + base, EPT)], col_v, sem2)

    zeros = jnp.zeros((16,), jnp.float32)

    @plsc.parallel_loop(0, NP // 16, unroll=8)
    def _zero(i):
        den_v[pl.ds(i * 16, 16)] = zeros

    d0.wait()
    d1.wait()
    d2.wait()

    @plsc.parallel_loop(0, VPT, unroll=8)
    def _stepa(i):
        sl = pl.ds(i * 16, 16)
        r = row_v[sl]
        cl = col_v[sl]
        a = plsc.load_gather(tab_v, [r])
        c = plsc.load_gather(tab_v, [cl + N_NODES])
        s = jnp.minimum(jnp.maximum(a + c, 0.0), 80.0)
        e = jnp.exp(s)
        ex_v[sl] = e
        plsc.addupdate_scatter(den_v, [r], e)

    pltpu.sync_copy(den_v, part_sh.at[pl.ds(sid * NP, NP)])
    plsc.subcore_barrier()

    cb = sid * CHUNK

    descs = [
        pltpu.async_copy(part_sh.at[pl.ds(j * NP + cb, CHUNK)],
                         tab_v.at[pl.ds(j * CHUNK, CHUNK)], sem0)
        for j in range(NSUB)
    ]
    for d in descs:
        d.wait()

    eps = jnp.float32(1e-16)

    @plsc.parallel_loop(0, CHUNK // 16, unroll=4)
    def _reduce(i):
        acc = tab_v[pl.ds(i * 16, 16)]

        def _acc(j, a_):
            return a_ + tab_v[pl.ds(j * CHUNK + i * 16, 16)]

        acc = lax.fori_loop(1, NSUB, _acc, acc)
        den_v[pl.ds(i * 16, 16)] = 1.0 / (acc + eps)

    pltpu.sync_copy(den_v.at[pl.ds(0, CHUNK)], inv_sh.at[pl.ds(cb, CHUNK)])
    plsc.subcore_barrier()
    pltpu.sync_copy(inv_sh, den_v)

    @plsc.parallel_loop(0, VPT, unroll=8)
    def _stepb(i):
        sl = pl.ds(i * 16, 16)
        inv = plsc.load_gather(den_v, [row_v[sl]])
        ex_v[sl] = ex_v[sl] * inv

    @pl.when(cid == 0)
    def _():
        pltpu.sync_copy(ex_v, v0_hbm.at[pl.ds(base, EPT)])

    @pl.when(cid == 1)
    def _():
        pltpu.sync_copy(ex_v, v1_hbm.at[pl.ds(base, EPT)])


@functools.partial(
    pl.kernel,
    out_type=(
        jax.ShapeDtypeStruct((N_EDGES,), jnp.float32),
        jax.ShapeDtypeStruct((N_EDGES,), jnp.float32),
    ),
    mesh=plsc.VectorSubcoreMesh(core_axis_name="c", subcore_axis_name="s"),
    scratch_types=[
        pltpu.VMEM((2 * N_NODES,), jnp.float32),
        pltpu.VMEM((EPT,), jnp.int32),
        pltpu.VMEM((EPT,), jnp.int32),
        pltpu.VMEM((EPT,), jnp.float32),
        pltpu.VMEM((NP,), jnp.float32),
        pltpu.VMEM_SHARED((NSUB * NP,), jnp.float32),
        pltpu.VMEM_SHARED((NP,), jnp.float32),
        pltpu.SemaphoreType.DMA,
        pltpu.SemaphoreType.DMA,
        pltpu.SemaphoreType.DMA,
    ],
    compiler_params=pltpu.CompilerParams(needs_layout_passes=False),
)
def _sc_fused(*refs):
    _sc_body(*refs)


def _combine_body(v0_ref, v1_ref, amt_ref, o_ref):
    o_ref[...] = jnp.where(amt_ref[...] != 0, v0_ref[...] + v1_ref[...], 0.0)


def _combine(v0, v1, amt):
    return pl.pallas_call(
        _combine_body,
        out_shape=jax.ShapeDtypeStruct((N_EDGES,), jnp.float32),
    )(v0, v1, amt)


def kernel(x, W, b, edge_index, actual_amount):
    eidx = edge_index.astype(jnp.int32).reshape(2 * N_EDGES)
    amt = actual_amount.astype(jnp.int32)

    tab = _projection(x, W, b).reshape(-1)

    v0, v1 = _sc_fused(tab, eidx)
    return _combine(v0, v1, amt)
